# baseline (speedup 1.0000x reference)
import functools

import jax
import jax.numpy as jnp
from jax import lax
from jax.experimental import pallas as pl
from jax.experimental.pallas import tpu as pltpu
from jax.experimental.pallas import tpu_sc as plsc

NC = 2
NS = 16
NW = NC * NS
LANES = 16

NBUF = 4


def _make_kernel(B, S, D, V):
    N = B * S
    seq_per_w = (N // S) // NW
    half = S // 2
    rounds = seq_per_w // NBUF

    mesh = plsc.VectorSubcoreMesh(core_axis_name="c", subcore_axis_name="s")

    @functools.partial(
        pl.kernel,
        out_type=jax.ShapeDtypeStruct((N, D), jnp.float32),
        mesh=mesh,
        compiler_params=pltpu.CompilerParams(use_tc_tiling_on_sc=False),
        scratch_types=[
            pltpu.VMEM((S, D), jnp.float32),
            pltpu.VMEM((NBUF, 2, half), jnp.int32),
            pltpu.VMEM((NBUF, S, D), jnp.float32),
            pltpu.SemaphoreType.DMA((NBUF,)),
            pltpu.SemaphoreType.DMA((NBUF,)),
            pltpu.SemaphoreType.DMA((NBUF,)),
        ],
    )
    def emb_kernel(idx_hbm, pos_hbm, table_hbm, out_hbm,
                   pos_v, idx_v, rows_v, si, sg, so):
        cid = lax.axis_index("c")
        sid = lax.axis_index("s")
        wid = sid * NC + cid
        base_seq = wid * seq_per_w

        pltpu.sync_copy(pos_hbm, pos_v)

        def idx_copy(seq, b):
            return pltpu.make_async_copy(
                idx_hbm.at[pl.ds(seq * 2, 2)], idx_v.at[b], si.at[b])

        def gather_copy(seq, b, j):
            return pltpu.make_async_copy(
                table_hbm.at[idx_v.at[b, j]],
                rows_v.at[b, pl.ds(j * half, half)],
                sg.at[b])

        def out_copy(seq, b):
            return pltpu.make_async_copy(
                rows_v.at[b], out_hbm.at[pl.ds(seq * S, S)], so.at[b])

        def add_pos(b):
            @plsc.parallel_loop(0, S, step=2, unroll=2)
            def _(r):
                for rr in (0, 1):
                    for c4 in range(D // LANES):
                        sl = pl.ds(c4 * LANES, LANES)
                        plsc.addupdate(rows_v.at[b, r + rr, sl],
                                       pos_v[r + rr, sl])

        def fire(o, b, first):
            seq = base_seq + o * NBUF + b
            idx_copy(seq, b).wait()
            if not first:
                out_copy(seq - NBUF, b).wait()
            gather_copy(seq, b, 0).start()
            gather_copy(seq, b, 1).start()

        def compute(o, b, last):
            seq = base_seq + o * NBUF + b
            gather_copy(seq, b, 0).wait()
            gather_copy(seq, b, 1).wait()
            if not last:
                idx_copy(seq + NBUF, b).start()
            add_pos(b)
            out_copy(seq, b).start()

        for b in range(NBUF):
            idx_copy(base_seq + b, b).start()

        for b in range(NBUF):
            fire(0, b, first=True)
        for b in range(NBUF):
            compute(0, b, last=False)

        def round_body(o, carry):
            for b in range(NBUF):
                fire(o, b, first=False)
            for b in range(NBUF):
                compute(o, b, last=False)
            return carry
        lax.fori_loop(1, rounds - 1, round_body, 0, unroll=False)

        o_last = rounds - 1
        for b in range(NBUF):
            fire(o_last, b, first=False)
        for b in range(NBUF):
            compute(o_last, b, last=True)

        for b in range(NBUF):
            out_copy(base_seq + o_last * NBUF + b, b).wait()

    return emb_kernel


def kernel(x, embedding, pos_encoding):
    B, S = x.shape
    V, D = embedding.shape
    idx2d = x.astype(jnp.int32).reshape(B * S // (S // 2), S // 2)
    out = _make_kernel(B, S, D, V)(idx2d, pos_encoding, embedding)
    return out.reshape(B, S, D)


# speedup vs baseline: 1.1746x; 1.1746x over previous
import functools

import jax
import jax.numpy as jnp
from jax import lax
from jax.experimental import pallas as pl
from jax.experimental.pallas import tpu as pltpu
from jax.experimental.pallas import tpu_sc as plsc

NC = 2
NS = 16
NW = NC * NS
LANES = 16
W = 128
CH = 40
NBUF = 8
WIN = 8
KIDX = 4
BLK = 32


def _make_kernel(B, S, D, V):
    N = B * S
    chunks_per_w = N // CH // NW
    blocks = chunks_per_w // BLK
    wins_per_w = chunks_per_w // WIN
    spc = S // CH
    bat_per_w = chunks_per_w // spc

    mesh = plsc.VectorSubcoreMesh(core_axis_name="c", subcore_axis_name="s")

    @functools.partial(
        pl.kernel,
        out_type=jax.ShapeDtypeStruct((B, S, D), jnp.float32),
        mesh=mesh,
        scratch_types=[
            pltpu.VMEM((S, D), jnp.float32),
            pltpu.VMEM((KIDX, WIN, CH), jnp.int32),
            pltpu.VMEM((NBUF, CH, W), jnp.float32),
            pltpu.VMEM((NBUF, CH, D), jnp.float32),
            pltpu.SemaphoreType.DMA((KIDX,)),
            pltpu.SemaphoreType.DMA((NBUF,)),
            pltpu.SemaphoreType.DMA((NBUF,)),
        ],
    )
    def emb_kernel(idx_hbm, pos_hbm, table_hbm, out_hbm,
                   pos_v, idx_v, rows_v, cout_v, si, sg, so):
        cid = lax.axis_index("c")
        sid = lax.axis_index("s")
        wid = sid * NC + cid

        pltpu.sync_copy(pos_hbm, pos_v)

        def idx_copy(w, k):
            return pltpu.make_async_copy(
                idx_hbm.at[pl.ds((wid * wins_per_w + w) * WIN, WIN)],
                idx_v.at[k], si.at[k])

        def gather_copy(c, crel):
            return pltpu.make_async_copy(
                table_hbm.at[idx_v.at[(crel // WIN) % KIDX, crel % WIN]],
                rows_v.at[crel % NBUF], sg.at[crel % NBUF])

        def out_copy(c, crel):
            b0 = wid * bat_per_w + c // spc
            return pltpu.make_async_copy(
                cout_v.at[crel % NBUF],
                out_hbm.at[b0, pl.ds((c % spc) * CH, CH)],
                so.at[crel % NBUF])

        def add_pos(c, crel):
            b = crel % NBUF
            p0 = (c % spc) * CH
            @plsc.parallel_loop(0, CH, step=2, unroll=1)
            def _(r):
                for rr in (0, 1):
                    for c4 in range(D // LANES):
                        sl = pl.ds(c4 * LANES, LANES)
                        cout_v[b, r + rr, sl] = (
                            rows_v[b, r + rr, sl] + pos_v[p0 + r + rr, sl])

        for k in range(KIDX):
            idx_copy(k, k).start()
        idx_copy(0, 0).wait()
        for crel in range(NBUF):
            gather_copy(crel, crel).start()

        def block_body(blk, carry):
            for crel in range(BLK):
                c = blk * BLK + crel
                gather_copy(c, crel).wait()

                @pl.when(c >= NBUF)
                def _():
                    out_copy(c - NBUF, (crel - NBUF) % BLK).wait()

                add_pos(c, crel)
                out_copy(c, crel).start()

                @pl.when(c + NBUF < chunks_per_w)
                def _():
                    if crel % WIN == 0:
                        idx_copy(c // WIN + 1,
                                 (crel // WIN + 1) % KIDX).wait()
                    gather_copy(c + NBUF, (crel + NBUF) % BLK).start()

                if crel % WIN == WIN - 1:
                    k = crel // WIN

                    @pl.when(blk < blocks - 1)
                    def _():
                        idx_copy(blk * KIDX + KIDX + k, k).start()
            return carry

        lax.fori_loop(0, blocks, block_body, 0, unroll=False)

        for crel in range(BLK - NBUF, BLK):
            out_copy((blocks - 1) * BLK + crel, crel).wait()

    return emb_kernel


def kernel(x, embedding, pos_encoding):
    B, S = x.shape
    V, D = embedding.shape
    idx40 = x.astype(jnp.int32).reshape(B * S // CH, CH)
    table_p = jnp.pad(embedding, ((0, 0), (0, W - D)))
    return _make_kernel(B, S, D, V)(idx40, pos_encoding, table_p)
